# baseline (speedup 1.0000x reference)
import functools

import jax
import jax.numpy as jnp
from jax import lax
from jax.experimental import pallas as pl
from jax.experimental.pallas import tpu as pltpu
from jax.experimental.pallas import tpu_sc as plsc

HOUR, WEEKDAY = 24, 7
NCOMB = HOUR * WEEKDAY
NCOMBP = 176
NC, NS = 2, 16
NW = NC * NS
CH = 128
NBUF = 5
NJ = CH // 128
W = 512
SC_ROWS = 409600


def _prep_kernel(hour_ref, wk_ref, w_ref, b_ref, comb_ref, hiT_ref, loT_ref):
    k_h = lax.broadcasted_iota(jnp.int32, (NCOMBP, HOUR), 0) // WEEKDAY
    i_h = lax.broadcasted_iota(jnp.int32, (NCOMBP, HOUR), 1)
    oh_h = (k_h == i_h).astype(jnp.float32)
    k_w = lax.broadcasted_iota(jnp.int32, (NCOMBP, 8), 0) % WEEKDAY
    j_w = lax.broadcasted_iota(jnp.int32, (NCOMBP, 8), 1)
    oh_w = (k_w == j_w).astype(jnp.float32)
    s = (lax.dot_general(oh_h, hour_ref[...], (((1,), (0,)), ((), ())),
                         preferred_element_type=jnp.float32)
         + lax.dot_general(oh_w, wk_ref[...], (((1,), (0,)), ((), ())),
                           preferred_element_type=jnp.float32))
    comb = lax.dot_general(s, w_ref[...], (((1,), (1,)), ((), ())),
                           preferred_element_type=jnp.float32) + b_ref[...]
    comb_ref[...] = comb[:NCOMB]
    combT = comb.T
    hiT = combT.astype(jnp.bfloat16)
    hiT_ref[...] = hiT
    loT_ref[...] = (combT - hiT.astype(jnp.float32)).astype(jnp.bfloat16)


def _tc_tail_kernel(alias_ref, x0_ref, x1_ref, hiT_ref, loT_ref, out_ref):
    del alias_ref
    c = x0_ref[0, 0, :] * WEEKDAY + x1_ref[0, 0, :]
    bc = jnp.broadcast_to(c.reshape(1, W), (NCOMBP, W))
    kio = lax.broadcasted_iota(jnp.int32, (NCOMBP, W), 0)
    oh = (bc == kio).astype(jnp.bfloat16)
    dn = (((1,), (0,)), ((), ()))
    outT = (lax.dot_general(hiT_ref[...], oh, dn,
                            preferred_element_type=jnp.float32)
            + lax.dot_general(loT_ref[...], oh, dn,
                              preferred_element_type=jnp.float32))
    out_ref[...] = outT.T


@functools.lru_cache(maxsize=None)
def _make_sc_gather(bl: int, sc_rows: int):
    per_tile = sc_rows // NW
    nch = per_tile // CH
    assert sc_rows % NW == 0 and per_tile % CH == 0 and nch % NBUF == 0

    mesh = plsc.VectorSubcoreMesh(core_axis_name="c", subcore_axis_name="s")

    @functools.partial(
        pl.kernel,
        mesh=mesh,
        out_type=jax.ShapeDtypeStruct((bl, 128), jnp.float32),
        scratch_types=(
            [pltpu.VMEM((NBUF, 2, CH), jnp.int32)]
            + [pltpu.VMEM((NBUF, NJ, 128), jnp.int32)]
            + [pltpu.VMEM((NBUF, CH, 128), jnp.float32)]
            + [pltpu.VMEM_SHARED((NCOMB, 128), jnp.float32)]
            + [pltpu.SemaphoreType.DMA] * (3 * NBUF)
        ),
    )
    def sc_gather(x0_hbm, x1_hbm, comb_hbm, out_hbm, xb, cv, rows, comb_v,
                  *sems):
        semx = sems[0:NBUF]
        semg = sems[NBUF:2 * NBUF]
        semw = sems[2 * NBUF:3 * NBUF]
        wid = lax.axis_index("c") * NS + lax.axis_index("s")
        base = wid * per_tile

        def xdescs(g, b):
            pos = base + g * CH
            return (
                pltpu.make_async_copy(x0_hbm.at[pl.ds(pos, CH)], xb.at[b, 0],
                                      semx[b]),
                pltpu.make_async_copy(x1_hbm.at[pl.ds(pos, CH)], xb.at[b, 1],
                                      semx[b]),
            )

        def gdescs(b):
            return [
                pltpu.make_async_copy(comb_v.at[cv.at[b, j]],
                                      rows.at[b, pl.ds(j * 128, 128)],
                                      semg[b])
                for j in range(NJ)
            ]

        def wdesc(g, b):
            pos = base + g * CH
            return pltpu.make_async_copy(rows.at[b], out_hbm.at[pl.ds(pos, CH)],
                                         semw[b])

        def chunk(g, b, first_round):
            d0, d1 = xdescs(g, b)
            d0.wait()
            d1.wait()
            for t in range(CH // 16):
                x0 = xb[b, 0, pl.ds(t * 16, 16)]
                x1 = xb[b, 1, pl.ds(t * 16, 16)]
                cv[b, t // 8, pl.ds((t % 8) * 16, 16)] = x0 * WEEKDAY + x1
            gx = jnp.minimum(g + NBUF, nch - 1)
            p0, p1 = xdescs(gx, b)
            p0.start()
            p1.start()
            if not first_round:
                wdesc(g, b).wait()
            for d in gdescs(b):
                d.start()
            if not (first_round and b == 0):
                pb = (b - 1) % NBUF
                for d in gdescs(pb):
                    d.wait()
                wdesc(g - 1, pb).start()

        @pl.when(lax.axis_index("s") == 0)
        def _():
            pltpu.sync_copy(comb_hbm, comb_v)
        plsc.subcore_barrier()

        for b in range(NBUF):
            d0, d1 = xdescs(b, b)
            d0.start()
            d1.start()
        for b in range(NBUF):
            chunk(b, b, first_round=True)

        def round_body(p, carry):
            for b in range(NBUF):
                chunk(p * NBUF + b, b, first_round=False)
            return carry

        lax.fori_loop(1, nch // NBUF, round_body, 0)

        last_b = (nch - 1) % NBUF
        for d in gdescs(last_b):
            d.wait()
        wdesc(nch - 1, last_b).start()
        for b in range(NBUF):
            wdesc(nch - 1, b).wait()
            d0, d1 = xdescs(nch - 1, b)
            d0.wait()
            d1.wait()

    return sc_gather


def kernel(x, hour_table, weekday_table, conv_w, conv_b):
    b, l, _ = x.shape
    d = hour_table.shape[1]
    bl = b * l
    x32 = x.astype(jnp.int32)
    wk8 = jnp.pad(weekday_table, ((0, 8 - WEEKDAY), (0, 0)))
    comb, hiT, loT = pl.pallas_call(
        _prep_kernel,
        out_shape=(
            jax.ShapeDtypeStruct((NCOMB, d), jnp.float32),
            jax.ShapeDtypeStruct((d, NCOMBP), jnp.bfloat16),
            jax.ShapeDtypeStruct((d, NCOMBP), jnp.bfloat16),
        ),
    )(hour_table, wk8, conv_w, conv_b.reshape(1, d))
    xt = x32.reshape(-1, 2).T
    x0, x1 = xt[0], xt[1]

    sc_rows = SC_ROWS
    sc_out = _make_sc_gather(bl, sc_rows)(x0, x1, comb)

    ntc = (bl - sc_rows) // W
    toff = sc_rows // W
    x0r = x0.reshape(-1, 1, W)
    x1r = x1.reshape(-1, 1, W)
    out = pl.pallas_call(
        _tc_tail_kernel,
        grid=(ntc,),
        in_specs=[
            pl.BlockSpec(memory_space=pl.ANY),
            pl.BlockSpec((1, 1, W), lambda i: (toff + i, 0, 0)),
            pl.BlockSpec((1, 1, W), lambda i: (toff + i, 0, 0)),
            pl.BlockSpec((d, NCOMBP), lambda i: (0, 0)),
            pl.BlockSpec((d, NCOMBP), lambda i: (0, 0)),
        ],
        out_specs=pl.BlockSpec((W, d), lambda i: (toff + i, 0)),
        out_shape=jax.ShapeDtypeStruct((bl, d), jnp.float32),
        input_output_aliases={},
    )(sc_out, x0r, x1r, hiT, loT)
    return out.reshape(b, l, d)


# speedup vs baseline: 1.8334x; 1.8334x over previous
import functools

import jax
import jax.numpy as jnp
from jax import lax
from jax.experimental import pallas as pl
from jax.experimental.pallas import tpu as pltpu
from jax.experimental.pallas import tpu_sc as plsc

HOUR, WEEKDAY = 24, 7
NCOMB = HOUR * WEEKDAY
NCOMBP = 176
NCAT = 40
KI = 4
NC, NS = 2, 16
NW = NC * NS
CH = 128
NBUF = 5
NJ = CH // 128
W = 512
SC_ROWS = 409600


def _prep_kernel(hour_ref, wk_ref, w_ref, b_ref, comb_ref, hiT_ref, loT_ref):
    k_h = lax.broadcasted_iota(jnp.int32, (NCOMBP, HOUR), 0) // WEEKDAY
    i_h = lax.broadcasted_iota(jnp.int32, (NCOMBP, HOUR), 1)
    oh_h = (k_h == i_h).astype(jnp.float32)
    k_w = lax.broadcasted_iota(jnp.int32, (NCOMBP, 8), 0) % WEEKDAY
    j_w = lax.broadcasted_iota(jnp.int32, (NCOMBP, 8), 1)
    oh_w = (k_w == j_w).astype(jnp.float32)
    s = (lax.dot_general(oh_h, hour_ref[...], (((1,), (0,)), ((), ())),
                         preferred_element_type=jnp.float32)
         + lax.dot_general(oh_w, wk_ref[...], (((1,), (0,)), ((), ())),
                           preferred_element_type=jnp.float32))
    comb = lax.dot_general(s, w_ref[...], (((1,), (1,)), ((), ())),
                           preferred_element_type=jnp.float32) + b_ref[...]
    comb_ref[...] = comb[:NCOMB]
    kc0 = lax.broadcasted_iota(jnp.int32, (NCAT, HOUR), 0)
    ic0 = lax.broadcasted_iota(jnp.int32, (NCAT, HOUR), 1)
    ohc_h = (kc0 == ic0).astype(jnp.float32)
    kc1 = lax.broadcasted_iota(jnp.int32, (NCAT, 8), 0) - 32
    jc1 = lax.broadcasted_iota(jnp.int32, (NCAT, 8), 1)
    ohc_w = (kc1 == jc1).astype(jnp.float32)
    s_cat = (lax.dot_general(ohc_h, hour_ref[...], (((1,), (0,)), ((), ())),
                             preferred_element_type=jnp.float32)
             + lax.dot_general(ohc_w, wk_ref[...], (((1,), (0,)), ((), ())),
                               preferred_element_type=jnp.float32))
    bmask = (lax.broadcasted_iota(jnp.int32, (NCAT, 128), 0) >= 32
             ).astype(jnp.float32)
    cat = (lax.dot_general(s_cat, w_ref[...], (((1,), (1,)), ((), ())),
                           preferred_element_type=jnp.float32)
           + bmask * b_ref[...])
    hi = cat.astype(jnp.bfloat16)
    hiT_ref[...] = hi
    loT_ref[...] = (cat - hi.astype(jnp.float32)).astype(jnp.bfloat16)


def _tc_tail_kernel(alias_ref, x0_ref, x1_ref, hiT_ref, loT_ref, out_ref):
    del alias_ref
    i40 = lax.broadcasted_iota(jnp.int32, (NCAT, W), 0)
    dn = (((0,), (0,)), ((), ()))
    for ki in range(KI):
        x0b = jnp.broadcast_to(x0_ref[0, ki, :].reshape(1, W), (NCAT, W))
        x1b = jnp.broadcast_to(x1_ref[0, ki, :].reshape(1, W), (NCAT, W))
        sel = jnp.where(i40 < 32, x0b, x1b + 32)
        oh = (sel == i40).astype(jnp.bfloat16)
        o = (lax.dot_general(oh, hiT_ref[...], dn,
                             preferred_element_type=jnp.float32)
             + lax.dot_general(oh, loT_ref[...], dn,
                               preferred_element_type=jnp.float32))
        out_ref[pl.ds(ki * W, W), :] = o


@functools.lru_cache(maxsize=None)
def _make_sc_gather(bl: int, sc_rows: int):
    per_tile = sc_rows // NW
    nch = per_tile // CH
    assert sc_rows % NW == 0 and per_tile % CH == 0 and nch % NBUF == 0

    mesh = plsc.VectorSubcoreMesh(core_axis_name="c", subcore_axis_name="s")

    @functools.partial(
        pl.kernel,
        mesh=mesh,
        out_type=jax.ShapeDtypeStruct((bl, 128), jnp.float32),
        scratch_types=(
            [pltpu.VMEM((NBUF, 2, CH), jnp.int32)]
            + [pltpu.VMEM((NBUF, NJ, 128), jnp.int32)]
            + [pltpu.VMEM((NBUF, CH, 128), jnp.float32)]
            + [pltpu.VMEM_SHARED((NCOMB, 128), jnp.float32)]
            + [pltpu.SemaphoreType.DMA] * (3 * NBUF)
        ),
    )
    def sc_gather(x0_hbm, x1_hbm, comb_hbm, out_hbm, xb, cv, rows, comb_v,
                  *sems):
        semx = sems[0:NBUF]
        semg = sems[NBUF:2 * NBUF]
        semw = sems[2 * NBUF:3 * NBUF]
        wid = lax.axis_index("c") * NS + lax.axis_index("s")
        base = wid * per_tile

        def xdescs(g, b):
            pos = base + g * CH
            return (
                pltpu.make_async_copy(x0_hbm.at[pl.ds(pos, CH)], xb.at[b, 0],
                                      semx[b]),
                pltpu.make_async_copy(x1_hbm.at[pl.ds(pos, CH)], xb.at[b, 1],
                                      semx[b]),
            )

        def gdescs(b):
            return [
                pltpu.make_async_copy(comb_v.at[cv.at[b, j]],
                                      rows.at[b, pl.ds(j * 128, 128)],
                                      semg[b])
                for j in range(NJ)
            ]

        def wdesc(g, b):
            pos = base + g * CH
            return pltpu.make_async_copy(rows.at[b], out_hbm.at[pl.ds(pos, CH)],
                                         semw[b])

        def chunk(g, b, first_round):
            d0, d1 = xdescs(g, b)
            d0.wait()
            d1.wait()
            for t in range(CH // 16):
                x0 = xb[b, 0, pl.ds(t * 16, 16)]
                x1 = xb[b, 1, pl.ds(t * 16, 16)]
                cv[b, t // 8, pl.ds((t % 8) * 16, 16)] = x0 * WEEKDAY + x1
            gx = jnp.minimum(g + NBUF, nch - 1)
            p0, p1 = xdescs(gx, b)
            p0.start()
            p1.start()
            if not first_round:
                wdesc(g, b).wait()
            for d in gdescs(b):
                d.start()
            if not (first_round and b == 0):
                pb = (b - 1) % NBUF
                for d in gdescs(pb):
                    d.wait()
                wdesc(g - 1, pb).start()

        @pl.when(lax.axis_index("s") == 0)
        def _():
            pltpu.sync_copy(comb_hbm, comb_v)
        plsc.subcore_barrier()

        for b in range(NBUF):
            d0, d1 = xdescs(b, b)
            d0.start()
            d1.start()
        for b in range(NBUF):
            chunk(b, b, first_round=True)

        def round_body(p, carry):
            for b in range(NBUF):
                chunk(p * NBUF + b, b, first_round=False)
            return carry

        lax.fori_loop(1, nch // NBUF, round_body, 0)

        last_b = (nch - 1) % NBUF
        for d in gdescs(last_b):
            d.wait()
        wdesc(nch - 1, last_b).start()
        for b in range(NBUF):
            wdesc(nch - 1, b).wait()
            d0, d1 = xdescs(nch - 1, b)
            d0.wait()
            d1.wait()

    return sc_gather


def kernel(x, hour_table, weekday_table, conv_w, conv_b):
    b, l, _ = x.shape
    d = hour_table.shape[1]
    bl = b * l
    x32 = x.astype(jnp.int32)
    wk8 = jnp.pad(weekday_table, ((0, 8 - WEEKDAY), (0, 0)))
    comb, hiT, loT = pl.pallas_call(
        _prep_kernel,
        out_shape=(
            jax.ShapeDtypeStruct((NCOMB, d), jnp.float32),
            jax.ShapeDtypeStruct((NCAT, d), jnp.bfloat16),
            jax.ShapeDtypeStruct((NCAT, d), jnp.bfloat16),
        ),
    )(hour_table, wk8, conv_w, conv_b.reshape(1, d))
    xt = x32.reshape(-1, 2).T
    x0, x1 = xt[0], xt[1]

    sc_rows = SC_ROWS
    sc_out = _make_sc_gather(bl, sc_rows)(x0, x1, comb)

    blk = KI * W
    ntc = (bl - sc_rows) // blk
    toff = sc_rows // blk
    x0r = x0.reshape(-1, KI, W)
    x1r = x1.reshape(-1, KI, W)
    out = pl.pallas_call(
        _tc_tail_kernel,
        grid=(ntc,),
        in_specs=[
            pl.BlockSpec(memory_space=pl.ANY),
            pl.BlockSpec((1, KI, W), lambda i: (toff + i, 0, 0)),
            pl.BlockSpec((1, KI, W), lambda i: (toff + i, 0, 0)),
            pl.BlockSpec((NCAT, d), lambda i: (0, 0)),
            pl.BlockSpec((NCAT, d), lambda i: (0, 0)),
        ],
        out_specs=pl.BlockSpec((blk, d), lambda i: (toff + i, 0)),
        out_shape=jax.ShapeDtypeStruct((bl, d), jnp.float32),
        input_output_aliases={0: 0},
    )(sc_out, x0r, x1r, hiT, loT)
    return out.reshape(b, l, d)


# speedup vs baseline: 2.1792x; 1.1886x over previous
import functools

import jax
import jax.numpy as jnp
from jax import lax
from jax.experimental import pallas as pl
from jax.experimental.pallas import tpu as pltpu
from jax.experimental.pallas import tpu_sc as plsc

HOUR, WEEKDAY = 24, 7
NCOMB = HOUR * WEEKDAY
NCOMBP = 176
NCAT = 40
KI = 8
NC, NS = 2, 16
NW = NC * NS
CH = 128
NBUF = 5
NJ = CH // 128
W = 512
SC_ROWS = 409600


def _prep_kernel(hour_ref, wk_ref, w_ref, b_ref, comb_ref, hiT_ref, loT_ref):
    k_h = lax.broadcasted_iota(jnp.int32, (NCOMBP, HOUR), 0) // WEEKDAY
    i_h = lax.broadcasted_iota(jnp.int32, (NCOMBP, HOUR), 1)
    oh_h = (k_h == i_h).astype(jnp.float32)
    k_w = lax.broadcasted_iota(jnp.int32, (NCOMBP, 8), 0) % WEEKDAY
    j_w = lax.broadcasted_iota(jnp.int32, (NCOMBP, 8), 1)
    oh_w = (k_w == j_w).astype(jnp.float32)
    s = (lax.dot_general(oh_h, hour_ref[...], (((1,), (0,)), ((), ())),
                         preferred_element_type=jnp.float32)
         + lax.dot_general(oh_w, wk_ref[...], (((1,), (0,)), ((), ())),
                           preferred_element_type=jnp.float32))
    comb = lax.dot_general(s, w_ref[...], (((1,), (1,)), ((), ())),
                           preferred_element_type=jnp.float32) + b_ref[...]
    comb_ref[...] = comb[:NCOMB]
    kc0 = lax.broadcasted_iota(jnp.int32, (NCAT, HOUR), 0)
    ic0 = lax.broadcasted_iota(jnp.int32, (NCAT, HOUR), 1)
    ohc_h = (kc0 == ic0).astype(jnp.float32)
    kc1 = lax.broadcasted_iota(jnp.int32, (NCAT, 8), 0) - 32
    jc1 = lax.broadcasted_iota(jnp.int32, (NCAT, 8), 1)
    ohc_w = (kc1 == jc1).astype(jnp.float32)
    s_cat = (lax.dot_general(ohc_h, hour_ref[...], (((1,), (0,)), ((), ())),
                             preferred_element_type=jnp.float32)
             + lax.dot_general(ohc_w, wk_ref[...], (((1,), (0,)), ((), ())),
                               preferred_element_type=jnp.float32))
    bmask = (lax.broadcasted_iota(jnp.int32, (NCAT, 128), 0) >= 32
             ).astype(jnp.float32)
    cat = (lax.dot_general(s_cat, w_ref[...], (((1,), (1,)), ((), ())),
                           preferred_element_type=jnp.float32)
           + bmask * b_ref[...])
    hi = cat.astype(jnp.bfloat16)
    hiT_ref[...] = hi
    loT_ref[...] = (cat - hi.astype(jnp.float32)).astype(jnp.bfloat16)


def _tc_tail_kernel(alias_ref, x0_ref, x1_ref, hiT_ref, loT_ref, out_ref):
    del alias_ref
    i40 = lax.broadcasted_iota(jnp.int32, (NCAT, W), 0)
    dn = (((0,), (0,)), ((), ()))
    for ki in range(KI):
        x0b = jnp.broadcast_to(x0_ref[0, ki, :].reshape(1, W), (NCAT, W))
        x1b = jnp.broadcast_to(x1_ref[0, ki, :].reshape(1, W), (NCAT, W))
        sel = jnp.where(i40 < 32, x0b, x1b + 32)
        oh = (sel == i40).astype(jnp.bfloat16)
        o = (lax.dot_general(oh, hiT_ref[...], dn,
                             preferred_element_type=jnp.float32)
             + lax.dot_general(oh, loT_ref[...], dn,
                               preferred_element_type=jnp.float32))
        out_ref[pl.ds(ki * W, W), :] = o


@functools.lru_cache(maxsize=None)
def _make_sc_gather(bl: int, sc_rows: int):
    per_tile = sc_rows // NW
    nch = per_tile // CH
    assert sc_rows % NW == 0 and per_tile % CH == 0 and nch % NBUF == 0

    mesh = plsc.VectorSubcoreMesh(core_axis_name="c", subcore_axis_name="s")

    @functools.partial(
        pl.kernel,
        mesh=mesh,
        out_type=jax.ShapeDtypeStruct((bl, 128), jnp.float32),
        scratch_types=(
            [pltpu.VMEM((NBUF, 2, CH), jnp.int32)]
            + [pltpu.VMEM((NBUF, NJ, 128), jnp.int32)]
            + [pltpu.VMEM((NBUF, CH, 128), jnp.float32)]
            + [pltpu.VMEM_SHARED((NCOMB, 128), jnp.float32)]
            + [pltpu.SemaphoreType.DMA] * (3 * NBUF)
        ),
    )
    def sc_gather(x0_hbm, x1_hbm, comb_hbm, out_hbm, xb, cv, rows, comb_v,
                  *sems):
        semx = sems[0:NBUF]
        semg = sems[NBUF:2 * NBUF]
        semw = sems[2 * NBUF:3 * NBUF]
        wid = lax.axis_index("c") * NS + lax.axis_index("s")
        base = wid * per_tile

        def xdescs(g, b):
            pos = base + g * CH
            return (
                pltpu.make_async_copy(x0_hbm.at[pl.ds(pos, CH)], xb.at[b, 0],
                                      semx[b]),
                pltpu.make_async_copy(x1_hbm.at[pl.ds(pos, CH)], xb.at[b, 1],
                                      semx[b]),
            )

        def gdescs(b):
            return [
                pltpu.make_async_copy(comb_v.at[cv.at[b, j]],
                                      rows.at[b, pl.ds(j * 128, 128)],
                                      semg[b])
                for j in range(NJ)
            ]

        def wdesc(g, b):
            pos = base + g * CH
            return pltpu.make_async_copy(rows.at[b], out_hbm.at[pl.ds(pos, CH)],
                                         semw[b])

        def chunk(g, b, first_round):
            d0, d1 = xdescs(g, b)
            d0.wait()
            d1.wait()
            for t in range(CH // 16):
                x0 = xb[b, 0, pl.ds(t * 16, 16)]
                x1 = xb[b, 1, pl.ds(t * 16, 16)]
                cv[b, t // 8, pl.ds((t % 8) * 16, 16)] = x0 * WEEKDAY + x1
            gx = jnp.minimum(g + NBUF, nch - 1)
            p0, p1 = xdescs(gx, b)
            p0.start()
            p1.start()
            if not first_round:
                wdesc(g, b).wait()
            for d in gdescs(b):
                d.start()
            if not (first_round and b == 0):
                pb = (b - 1) % NBUF
                for d in gdescs(pb):
                    d.wait()
                wdesc(g - 1, pb).start()

        @pl.when(lax.axis_index("s") == 0)
        def _():
            pltpu.sync_copy(comb_hbm, comb_v)
        plsc.subcore_barrier()

        for b in range(NBUF):
            d0, d1 = xdescs(b, b)
            d0.start()
            d1.start()
        for b in range(NBUF):
            chunk(b, b, first_round=True)

        def round_body(p, carry):
            for b in range(NBUF):
                chunk(p * NBUF + b, b, first_round=False)
            return carry

        lax.fori_loop(1, nch // NBUF, round_body, 0)

        last_b = (nch - 1) % NBUF
        for d in gdescs(last_b):
            d.wait()
        wdesc(nch - 1, last_b).start()
        for b in range(NBUF):
            wdesc(nch - 1, b).wait()
            d0, d1 = xdescs(nch - 1, b)
            d0.wait()
            d1.wait()

    return sc_gather


def kernel(x, hour_table, weekday_table, conv_w, conv_b):
    b, l, _ = x.shape
    d = hour_table.shape[1]
    bl = b * l
    x32 = x.astype(jnp.int32)
    wk8 = jnp.pad(weekday_table, ((0, 8 - WEEKDAY), (0, 0)))
    comb, hiT, loT = pl.pallas_call(
        _prep_kernel,
        out_shape=(
            jax.ShapeDtypeStruct((NCOMB, d), jnp.float32),
            jax.ShapeDtypeStruct((NCAT, d), jnp.bfloat16),
            jax.ShapeDtypeStruct((NCAT, d), jnp.bfloat16),
        ),
    )(hour_table, wk8, conv_w, conv_b.reshape(1, d))
    xt = x32.reshape(-1, 2).T
    x0, x1 = xt[0], xt[1]

    sc_rows = SC_ROWS
    sc_out = _make_sc_gather(bl, sc_rows)(x0, x1, comb)

    blk = KI * W
    ntc = (bl - sc_rows) // blk
    toff = sc_rows // blk
    x0r = x0.reshape(-1, KI, W)
    x1r = x1.reshape(-1, KI, W)
    out = pl.pallas_call(
        _tc_tail_kernel,
        grid=(ntc,),
        in_specs=[
            pl.BlockSpec(memory_space=pl.ANY),
            pl.BlockSpec((1, KI, W), lambda i: (toff + i, 0, 0)),
            pl.BlockSpec((1, KI, W), lambda i: (toff + i, 0, 0)),
            pl.BlockSpec((NCAT, d), lambda i: (0, 0)),
            pl.BlockSpec((NCAT, d), lambda i: (0, 0)),
        ],
        out_specs=pl.BlockSpec((blk, d), lambda i: (toff + i, 0)),
        out_shape=jax.ShapeDtypeStruct((bl, d), jnp.float32),
        input_output_aliases={0: 0},
    )(sc_out, x0r, x1r, hiT, loT)
    return out.reshape(b, l, d)


# speedup vs baseline: 2.4079x; 1.1050x over previous
import functools

import jax
import jax.numpy as jnp
from jax import lax
from jax.experimental import pallas as pl
from jax.experimental.pallas import tpu as pltpu
from jax.experimental.pallas import tpu_sc as plsc

HOUR, WEEKDAY = 24, 7
NCOMB = HOUR * WEEKDAY
NCOMBP = 176
NCAT = 40
KI = 16
NC, NS = 2, 16
NW = NC * NS
CH = 128
NBUF = 5
NJ = CH // 128
W = 512
SC_ROWS = 409600


def _prep_kernel(hour_ref, wk_ref, w_ref, b_ref, comb_ref, hiT_ref, loT_ref):
    k_h = lax.broadcasted_iota(jnp.int32, (NCOMBP, HOUR), 0) // WEEKDAY
    i_h = lax.broadcasted_iota(jnp.int32, (NCOMBP, HOUR), 1)
    oh_h = (k_h == i_h).astype(jnp.float32)
    k_w = lax.broadcasted_iota(jnp.int32, (NCOMBP, 8), 0) % WEEKDAY
    j_w = lax.broadcasted_iota(jnp.int32, (NCOMBP, 8), 1)
    oh_w = (k_w == j_w).astype(jnp.float32)
    s = (lax.dot_general(oh_h, hour_ref[...], (((1,), (0,)), ((), ())),
                         preferred_element_type=jnp.float32)
         + lax.dot_general(oh_w, wk_ref[...], (((1,), (0,)), ((), ())),
                           preferred_element_type=jnp.float32))
    comb = lax.dot_general(s, w_ref[...], (((1,), (1,)), ((), ())),
                           preferred_element_type=jnp.float32) + b_ref[...]
    comb_ref[...] = comb[:NCOMB]
    kc0 = lax.broadcasted_iota(jnp.int32, (NCAT, HOUR), 0)
    ic0 = lax.broadcasted_iota(jnp.int32, (NCAT, HOUR), 1)
    ohc_h = (kc0 == ic0).astype(jnp.float32)
    kc1 = lax.broadcasted_iota(jnp.int32, (NCAT, 8), 0) - 32
    jc1 = lax.broadcasted_iota(jnp.int32, (NCAT, 8), 1)
    ohc_w = (kc1 == jc1).astype(jnp.float32)
    s_cat = (lax.dot_general(ohc_h, hour_ref[...], (((1,), (0,)), ((), ())),
                             preferred_element_type=jnp.float32)
             + lax.dot_general(ohc_w, wk_ref[...], (((1,), (0,)), ((), ())),
                               preferred_element_type=jnp.float32))
    bmask = (lax.broadcasted_iota(jnp.int32, (NCAT, 128), 0) >= 32
             ).astype(jnp.float32)
    cat = (lax.dot_general(s_cat, w_ref[...], (((1,), (1,)), ((), ())),
                           preferred_element_type=jnp.float32)
           + bmask * b_ref[...])
    hi = cat.astype(jnp.bfloat16)
    hiT_ref[...] = hi
    loT_ref[...] = (cat - hi.astype(jnp.float32)).astype(jnp.bfloat16)


def _tc_tail_kernel(alias_ref, x0_ref, x1_ref, hiT_ref, loT_ref, out_ref):
    del alias_ref
    i40 = lax.broadcasted_iota(jnp.int32, (NCAT, W), 0)
    dn = (((0,), (0,)), ((), ()))
    for ki in range(KI):
        x0b = jnp.broadcast_to(x0_ref[0, ki, :].reshape(1, W), (NCAT, W))
        x1b = jnp.broadcast_to(x1_ref[0, ki, :].reshape(1, W), (NCAT, W))
        sel = jnp.where(i40 < 32, x0b, x1b + 32)
        oh = (sel == i40).astype(jnp.bfloat16)
        o = (lax.dot_general(oh, hiT_ref[...], dn,
                             preferred_element_type=jnp.float32)
             + lax.dot_general(oh, loT_ref[...], dn,
                               preferred_element_type=jnp.float32))
        out_ref[pl.ds(ki * W, W), :] = o


@functools.lru_cache(maxsize=None)
def _make_sc_gather(bl: int, sc_rows: int):
    per_tile = sc_rows // NW
    nch = per_tile // CH
    assert sc_rows % NW == 0 and per_tile % CH == 0 and nch % NBUF == 0

    mesh = plsc.VectorSubcoreMesh(core_axis_name="c", subcore_axis_name="s")

    @functools.partial(
        pl.kernel,
        mesh=mesh,
        out_type=jax.ShapeDtypeStruct((bl, 128), jnp.float32),
        scratch_types=(
            [pltpu.VMEM((NBUF, 2, CH), jnp.int32)]
            + [pltpu.VMEM((NBUF, NJ, 128), jnp.int32)]
            + [pltpu.VMEM((NBUF, CH, 128), jnp.float32)]
            + [pltpu.VMEM_SHARED((NCOMB, 128), jnp.float32)]
            + [pltpu.SemaphoreType.DMA] * (3 * NBUF)
        ),
    )
    def sc_gather(x0_hbm, x1_hbm, comb_hbm, out_hbm, xb, cv, rows, comb_v,
                  *sems):
        semx = sems[0:NBUF]
        semg = sems[NBUF:2 * NBUF]
        semw = sems[2 * NBUF:3 * NBUF]
        wid = lax.axis_index("c") * NS + lax.axis_index("s")
        base = wid * per_tile

        def xdescs(g, b):
            pos = base + g * CH
            return (
                pltpu.make_async_copy(x0_hbm.at[pl.ds(pos, CH)], xb.at[b, 0],
                                      semx[b]),
                pltpu.make_async_copy(x1_hbm.at[pl.ds(pos, CH)], xb.at[b, 1],
                                      semx[b]),
            )

        def gdescs(b):
            return [
                pltpu.make_async_copy(comb_v.at[cv.at[b, j]],
                                      rows.at[b, pl.ds(j * 128, 128)],
                                      semg[b])
                for j in range(NJ)
            ]

        def wdesc(g, b):
            pos = base + g * CH
            return pltpu.make_async_copy(rows.at[b], out_hbm.at[pl.ds(pos, CH)],
                                         semw[b])

        def chunk(g, b, first_round):
            d0, d1 = xdescs(g, b)
            d0.wait()
            d1.wait()
            for t in range(CH // 16):
                x0 = xb[b, 0, pl.ds(t * 16, 16)]
                x1 = xb[b, 1, pl.ds(t * 16, 16)]
                cv[b, t // 8, pl.ds((t % 8) * 16, 16)] = x0 * WEEKDAY + x1
            gx = jnp.minimum(g + NBUF, nch - 1)
            p0, p1 = xdescs(gx, b)
            p0.start()
            p1.start()
            if not first_round:
                wdesc(g, b).wait()
            for d in gdescs(b):
                d.start()
            if not (first_round and b == 0):
                pb = (b - 1) % NBUF
                for d in gdescs(pb):
                    d.wait()
                wdesc(g - 1, pb).start()

        @pl.when(lax.axis_index("s") == 0)
        def _():
            pltpu.sync_copy(comb_hbm, comb_v)
        plsc.subcore_barrier()

        for b in range(NBUF):
            d0, d1 = xdescs(b, b)
            d0.start()
            d1.start()
        for b in range(NBUF):
            chunk(b, b, first_round=True)

        def round_body(p, carry):
            for b in range(NBUF):
                chunk(p * NBUF + b, b, first_round=False)
            return carry

        lax.fori_loop(1, nch // NBUF, round_body, 0)

        last_b = (nch - 1) % NBUF
        for d in gdescs(last_b):
            d.wait()
        wdesc(nch - 1, last_b).start()
        for b in range(NBUF):
            wdesc(nch - 1, b).wait()
            d0, d1 = xdescs(nch - 1, b)
            d0.wait()
            d1.wait()

    return sc_gather


def kernel(x, hour_table, weekday_table, conv_w, conv_b):
    b, l, _ = x.shape
    d = hour_table.shape[1]
    bl = b * l
    x32 = x.astype(jnp.int32)
    wk8 = jnp.pad(weekday_table, ((0, 8 - WEEKDAY), (0, 0)))
    comb, hiT, loT = pl.pallas_call(
        _prep_kernel,
        out_shape=(
            jax.ShapeDtypeStruct((NCOMB, d), jnp.float32),
            jax.ShapeDtypeStruct((NCAT, d), jnp.bfloat16),
            jax.ShapeDtypeStruct((NCAT, d), jnp.bfloat16),
        ),
    )(hour_table, wk8, conv_w, conv_b.reshape(1, d))
    xt = x32.reshape(-1, 2).T
    x0, x1 = xt[0], xt[1]

    sc_rows = SC_ROWS
    sc_out = _make_sc_gather(bl, sc_rows)(x0, x1, comb)

    blk = KI * W
    ntc = (bl - sc_rows) // blk
    toff = sc_rows // blk
    x0r = x0.reshape(-1, KI, W)
    x1r = x1.reshape(-1, KI, W)
    out = pl.pallas_call(
        _tc_tail_kernel,
        grid=(ntc,),
        in_specs=[
            pl.BlockSpec(memory_space=pl.ANY),
            pl.BlockSpec((1, KI, W), lambda i: (toff + i, 0, 0)),
            pl.BlockSpec((1, KI, W), lambda i: (toff + i, 0, 0)),
            pl.BlockSpec((NCAT, d), lambda i: (0, 0)),
            pl.BlockSpec((NCAT, d), lambda i: (0, 0)),
        ],
        out_specs=pl.BlockSpec((blk, d), lambda i: (toff + i, 0)),
        out_shape=jax.ShapeDtypeStruct((bl, d), jnp.float32),
        input_output_aliases={0: 0},
    )(sc_out, x0r, x1r, hiT, loT)
    return out.reshape(b, l, d)


# speedup vs baseline: 2.4917x; 1.0348x over previous
import functools

import jax
import jax.numpy as jnp
from jax import lax
from jax.experimental import pallas as pl
from jax.experimental.pallas import tpu as pltpu
from jax.experimental.pallas import tpu_sc as plsc

HOUR, WEEKDAY = 24, 7
NCOMB = HOUR * WEEKDAY
NCOMBP = 176
NCAT = 40
KI = 32
NC, NS = 2, 16
NW = NC * NS
CH = 128
NBUF = 5
NJ = CH // 128
W = 512
SC_ROWS = 409600


def _prep_kernel(hour_ref, wk_ref, w_ref, b_ref, comb_ref, hiT_ref, loT_ref):
    k_h = lax.broadcasted_iota(jnp.int32, (NCOMBP, HOUR), 0) // WEEKDAY
    i_h = lax.broadcasted_iota(jnp.int32, (NCOMBP, HOUR), 1)
    oh_h = (k_h == i_h).astype(jnp.float32)
    k_w = lax.broadcasted_iota(jnp.int32, (NCOMBP, 8), 0) % WEEKDAY
    j_w = lax.broadcasted_iota(jnp.int32, (NCOMBP, 8), 1)
    oh_w = (k_w == j_w).astype(jnp.float32)
    s = (lax.dot_general(oh_h, hour_ref[...], (((1,), (0,)), ((), ())),
                         preferred_element_type=jnp.float32)
         + lax.dot_general(oh_w, wk_ref[...], (((1,), (0,)), ((), ())),
                           preferred_element_type=jnp.float32))
    comb = lax.dot_general(s, w_ref[...], (((1,), (1,)), ((), ())),
                           preferred_element_type=jnp.float32) + b_ref[...]
    comb_ref[...] = comb[:NCOMB]
    kc0 = lax.broadcasted_iota(jnp.int32, (NCAT, HOUR), 0)
    ic0 = lax.broadcasted_iota(jnp.int32, (NCAT, HOUR), 1)
    ohc_h = (kc0 == ic0).astype(jnp.float32)
    kc1 = lax.broadcasted_iota(jnp.int32, (NCAT, 8), 0) - 32
    jc1 = lax.broadcasted_iota(jnp.int32, (NCAT, 8), 1)
    ohc_w = (kc1 == jc1).astype(jnp.float32)
    s_cat = (lax.dot_general(ohc_h, hour_ref[...], (((1,), (0,)), ((), ())),
                             preferred_element_type=jnp.float32)
             + lax.dot_general(ohc_w, wk_ref[...], (((1,), (0,)), ((), ())),
                               preferred_element_type=jnp.float32))
    bmask = (lax.broadcasted_iota(jnp.int32, (NCAT, 128), 0) >= 32
             ).astype(jnp.float32)
    cat = (lax.dot_general(s_cat, w_ref[...], (((1,), (1,)), ((), ())),
                           preferred_element_type=jnp.float32)
           + bmask * b_ref[...])
    hi = cat.astype(jnp.bfloat16)
    hiT_ref[...] = hi
    loT_ref[...] = (cat - hi.astype(jnp.float32)).astype(jnp.bfloat16)


def _tc_tail_kernel(alias_ref, x0_ref, x1_ref, hiT_ref, loT_ref, out_ref):
    del alias_ref
    i40 = lax.broadcasted_iota(jnp.int32, (NCAT, W), 0)
    dn = (((0,), (0,)), ((), ()))
    for ki in range(KI):
        x0b = jnp.broadcast_to(x0_ref[0, ki, :].reshape(1, W), (NCAT, W))
        x1b = jnp.broadcast_to(x1_ref[0, ki, :].reshape(1, W), (NCAT, W))
        sel = jnp.where(i40 < 32, x0b, x1b + 32)
        oh = (sel == i40).astype(jnp.bfloat16)
        o = (lax.dot_general(oh, hiT_ref[...], dn,
                             preferred_element_type=jnp.float32)
             + lax.dot_general(oh, loT_ref[...], dn,
                               preferred_element_type=jnp.float32))
        out_ref[pl.ds(ki * W, W), :] = o


@functools.lru_cache(maxsize=None)
def _make_sc_gather(bl: int, sc_rows: int):
    per_tile = sc_rows // NW
    nch = per_tile // CH
    assert sc_rows % NW == 0 and per_tile % CH == 0 and nch % NBUF == 0

    mesh = plsc.VectorSubcoreMesh(core_axis_name="c", subcore_axis_name="s")

    @functools.partial(
        pl.kernel,
        mesh=mesh,
        out_type=jax.ShapeDtypeStruct((bl, 128), jnp.float32),
        scratch_types=(
            [pltpu.VMEM((NBUF, 2, CH), jnp.int32)]
            + [pltpu.VMEM((NBUF, NJ, 128), jnp.int32)]
            + [pltpu.VMEM((NBUF, CH, 128), jnp.float32)]
            + [pltpu.VMEM_SHARED((NCOMB, 128), jnp.float32)]
            + [pltpu.SemaphoreType.DMA] * (3 * NBUF)
        ),
    )
    def sc_gather(x0_hbm, x1_hbm, comb_hbm, out_hbm, xb, cv, rows, comb_v,
                  *sems):
        semx = sems[0:NBUF]
        semg = sems[NBUF:2 * NBUF]
        semw = sems[2 * NBUF:3 * NBUF]
        wid = lax.axis_index("c") * NS + lax.axis_index("s")
        base = wid * per_tile

        def xdescs(g, b):
            pos = base + g * CH
            return (
                pltpu.make_async_copy(x0_hbm.at[pl.ds(pos, CH)], xb.at[b, 0],
                                      semx[b]),
                pltpu.make_async_copy(x1_hbm.at[pl.ds(pos, CH)], xb.at[b, 1],
                                      semx[b]),
            )

        def gdescs(b):
            return [
                pltpu.make_async_copy(comb_v.at[cv.at[b, j]],
                                      rows.at[b, pl.ds(j * 128, 128)],
                                      semg[b])
                for j in range(NJ)
            ]

        def wdesc(g, b):
            pos = base + g * CH
            return pltpu.make_async_copy(rows.at[b], out_hbm.at[pl.ds(pos, CH)],
                                         semw[b])

        def chunk(g, b, first_round):
            d0, d1 = xdescs(g, b)
            d0.wait()
            d1.wait()
            for t in range(CH // 16):
                x0 = xb[b, 0, pl.ds(t * 16, 16)]
                x1 = xb[b, 1, pl.ds(t * 16, 16)]
                cv[b, t // 8, pl.ds((t % 8) * 16, 16)] = x0 * WEEKDAY + x1
            gx = jnp.minimum(g + NBUF, nch - 1)
            p0, p1 = xdescs(gx, b)
            p0.start()
            p1.start()
            if not first_round:
                wdesc(g, b).wait()
            for d in gdescs(b):
                d.start()
            if not (first_round and b == 0):
                pb = (b - 1) % NBUF
                for d in gdescs(pb):
                    d.wait()
                wdesc(g - 1, pb).start()

        @pl.when(lax.axis_index("s") == 0)
        def _():
            pltpu.sync_copy(comb_hbm, comb_v)
        plsc.subcore_barrier()

        for b in range(NBUF):
            d0, d1 = xdescs(b, b)
            d0.start()
            d1.start()
        for b in range(NBUF):
            chunk(b, b, first_round=True)

        def round_body(p, carry):
            for b in range(NBUF):
                chunk(p * NBUF + b, b, first_round=False)
            return carry

        lax.fori_loop(1, nch // NBUF, round_body, 0)

        last_b = (nch - 1) % NBUF
        for d in gdescs(last_b):
            d.wait()
        wdesc(nch - 1, last_b).start()
        for b in range(NBUF):
            wdesc(nch - 1, b).wait()
            d0, d1 = xdescs(nch - 1, b)
            d0.wait()
            d1.wait()

    return sc_gather


def kernel(x, hour_table, weekday_table, conv_w, conv_b):
    b, l, _ = x.shape
    d = hour_table.shape[1]
    bl = b * l
    x32 = x.astype(jnp.int32)
    wk8 = jnp.pad(weekday_table, ((0, 8 - WEEKDAY), (0, 0)))
    comb, hiT, loT = pl.pallas_call(
        _prep_kernel,
        out_shape=(
            jax.ShapeDtypeStruct((NCOMB, d), jnp.float32),
            jax.ShapeDtypeStruct((NCAT, d), jnp.bfloat16),
            jax.ShapeDtypeStruct((NCAT, d), jnp.bfloat16),
        ),
    )(hour_table, wk8, conv_w, conv_b.reshape(1, d))
    xt = x32.reshape(-1, 2).T
    x0, x1 = xt[0], xt[1]

    sc_rows = SC_ROWS
    sc_out = _make_sc_gather(bl, sc_rows)(x0, x1, comb)

    blk = KI * W
    ntc = (bl - sc_rows) // blk
    toff = sc_rows // blk
    x0r = x0.reshape(-1, KI, W)
    x1r = x1.reshape(-1, KI, W)
    out = pl.pallas_call(
        _tc_tail_kernel,
        grid=(ntc,),
        in_specs=[
            pl.BlockSpec(memory_space=pl.ANY),
            pl.BlockSpec((1, KI, W), lambda i: (toff + i, 0, 0)),
            pl.BlockSpec((1, KI, W), lambda i: (toff + i, 0, 0)),
            pl.BlockSpec((NCAT, d), lambda i: (0, 0)),
            pl.BlockSpec((NCAT, d), lambda i: (0, 0)),
        ],
        out_specs=pl.BlockSpec((blk, d), lambda i: (toff + i, 0)),
        out_shape=jax.ShapeDtypeStruct((bl, d), jnp.float32),
        input_output_aliases={0: 0},
    )(sc_out, x0r, x1r, hiT, loT)
    return out.reshape(b, l, d)


# speedup vs baseline: 2.9023x; 1.1648x over previous
import functools

import jax
import jax.numpy as jnp
from jax import lax
from jax.experimental import pallas as pl
from jax.experimental.pallas import tpu as pltpu
from jax.experimental.pallas import tpu_sc as plsc

HOUR, WEEKDAY = 24, 7
NCOMB = HOUR * WEEKDAY
NC, NS = 2, 16
NW = NC * NS
CH = 128
NBUF = 5
NJ = CH // 128


def _prep_kernel(hour_ref, wk_ref, w_ref, b_ref, comb_ref):
    k_h = lax.broadcasted_iota(jnp.int32, (NCOMB, HOUR), 0) // WEEKDAY
    i_h = lax.broadcasted_iota(jnp.int32, (NCOMB, HOUR), 1)
    oh_h = (k_h == i_h).astype(jnp.float32)
    k_w = lax.broadcasted_iota(jnp.int32, (NCOMB, 8), 0) % WEEKDAY
    j_w = lax.broadcasted_iota(jnp.int32, (NCOMB, 8), 1)
    oh_w = (k_w == j_w).astype(jnp.float32)
    s = (lax.dot_general(oh_h, hour_ref[...], (((1,), (0,)), ((), ())),
                         preferred_element_type=jnp.float32)
         + lax.dot_general(oh_w, wk_ref[...], (((1,), (0,)), ((), ())),
                           preferred_element_type=jnp.float32))
    comb_ref[...] = lax.dot_general(
        s, w_ref[...], (((1,), (1,)), ((), ())),
        preferred_element_type=jnp.float32) + b_ref[...]


@functools.lru_cache(maxsize=None)
def _make_sc_gather(bl: int):
    per_tile = bl // NW
    nch = per_tile // CH
    assert bl % NW == 0 and per_tile % CH == 0 and nch % NBUF == 0

    mesh = plsc.VectorSubcoreMesh(core_axis_name="c", subcore_axis_name="s")

    @functools.partial(
        pl.kernel,
        mesh=mesh,
        out_type=jax.ShapeDtypeStruct((bl, 128), jnp.float32),
        scratch_types=(
            [pltpu.VMEM((NBUF, 2, CH), jnp.int32)]
            + [pltpu.VMEM((NBUF, NJ, 128), jnp.int32)]
            + [pltpu.VMEM((NBUF, CH, 128), jnp.float32)]
            + [pltpu.VMEM_SHARED((NCOMB, 128), jnp.float32)]
            + [pltpu.SemaphoreType.DMA] * (3 * NBUF)
        ),
    )
    def sc_gather(x0_hbm, x1_hbm, comb_hbm, out_hbm, xb, cv, rows, comb_v,
                  *sems):
        semx = sems[0:NBUF]
        semg = sems[NBUF:2 * NBUF]
        semw = sems[2 * NBUF:3 * NBUF]
        wid = lax.axis_index("c") * NS + lax.axis_index("s")
        base = wid * per_tile

        def xdescs(g, b):
            pos = base + g * CH
            return (
                pltpu.make_async_copy(x0_hbm.at[pl.ds(pos, CH)], xb.at[b, 0],
                                      semx[b]),
                pltpu.make_async_copy(x1_hbm.at[pl.ds(pos, CH)], xb.at[b, 1],
                                      semx[b]),
            )

        def gdescs(b):
            return [
                pltpu.make_async_copy(comb_v.at[cv.at[b, j]],
                                      rows.at[b, pl.ds(j * 128, 128)],
                                      semg[b])
                for j in range(NJ)
            ]

        def wdesc(g, b):
            pos = base + g * CH
            return pltpu.make_async_copy(rows.at[b], out_hbm.at[pl.ds(pos, CH)],
                                         semw[b])

        def chunk(g, b, first_round):
            d0, d1 = xdescs(g, b)
            d0.wait()
            d1.wait()
            for t in range(CH // 16):
                x0 = xb[b, 0, pl.ds(t * 16, 16)]
                x1 = xb[b, 1, pl.ds(t * 16, 16)]
                cv[b, t // 8, pl.ds((t % 8) * 16, 16)] = x0 * WEEKDAY + x1
            gx = jnp.minimum(g + NBUF, nch - 1)
            p0, p1 = xdescs(gx, b)
            p0.start()
            p1.start()
            if not first_round:
                wdesc(g, b).wait()
            for d in gdescs(b):
                d.start()
            if not (first_round and b == 0):
                pb = (b - 1) % NBUF
                for d in gdescs(pb):
                    d.wait()
                wdesc(g - 1, pb).start()

        @pl.when(lax.axis_index("s") == 0)
        def _():
            pltpu.sync_copy(comb_hbm, comb_v)
        plsc.subcore_barrier()

        for b in range(NBUF):
            d0, d1 = xdescs(b, b)
            d0.start()
            d1.start()
        for b in range(NBUF):
            chunk(b, b, first_round=True)

        def round_body(p, carry):
            for b in range(NBUF):
                chunk(p * NBUF + b, b, first_round=False)
            return carry

        lax.fori_loop(1, nch // NBUF, round_body, 0)

        last_b = (nch - 1) % NBUF
        for d in gdescs(last_b):
            d.wait()
        wdesc(nch - 1, last_b).start()
        for b in range(NBUF):
            wdesc(nch - 1, b).wait()
            d0, d1 = xdescs(nch - 1, b)
            d0.wait()
            d1.wait()

    return sc_gather


def kernel(x, hour_table, weekday_table, conv_w, conv_b):
    b, l, _ = x.shape
    d = hour_table.shape[1]
    x32 = x.astype(jnp.int32)
    wk8 = jnp.pad(weekday_table, ((0, 8 - WEEKDAY), (0, 0)))
    comb = pl.pallas_call(
        _prep_kernel,
        out_shape=jax.ShapeDtypeStruct((NCOMB, d), jnp.float32),
    )(hour_table, wk8, conv_w, conv_b.reshape(1, d))
    xt = x32.reshape(-1, 2).T
    out = _make_sc_gather(b * l)(xt[0], xt[1], comb)
    return out.reshape(b, l, d)
